# baseline (speedup 1.0000x reference)
import math

import jax
import jax.numpy as jnp
from jax.experimental import pallas as pl
from jax.experimental.pallas import tpu as pltpu

_N = 5000
_C = 91
_CF = _C - 1
_K = 100
_SUB = 8
_LN = 640
_NP = _SUB * _LN
_CLIP = math.log(1000.0 / 16.0)
_SCORE_THRESH = 0.05
_NMS_THRESH = 0.5
_NEG_INF = float("-inf")


def _nms_body(lg_ref, br_ref, pr_ref, hw_ref,
              obf_ref, olab_ref,
              ms_ref, ox1_ref, oy1_ref, ox2_ref, oy2_ref, ar_ref, cm_ref):
    h = hw_ref[0, 0]
    w = hw_ref[0, 1]

    px1 = pr_ref[0]
    py1 = pr_ref[1]
    px2 = pr_ref[2]
    py2 = pr_ref[3]
    widths = px2 - px1
    heights = py2 - py1
    ctr_x = px1 + 0.5 * widths
    ctr_y = py1 + 0.5 * heights

    l90 = lg_ref[1:91]
    l0 = lg_ref[0]
    mx = jnp.maximum(jnp.max(l90, axis=0), l0)
    e90 = jnp.exp(l90 - mx)
    denom = jnp.sum(e90, axis=0) + jnp.exp(l0 - mx)
    sc = e90 / denom

    dx = br_ref[1:91, 0] / 10.0
    dy = br_ref[1:91, 1] / 10.0
    dw = jnp.minimum(br_ref[1:91, 2] / 5.0, _CLIP)
    dh = jnp.minimum(br_ref[1:91, 3] / 5.0, _CLIP)
    pcx = dx * widths + ctr_x
    pcy = dy * heights + ctr_y
    pw = jnp.exp(dw) * widths
    ph = jnp.exp(dh) * heights
    x1 = jnp.clip(pcx - 0.5 * pw, 0.0, w)
    y1 = jnp.clip(pcy - 0.5 * ph, 0.0, h)
    x2 = jnp.clip(pcx + 0.5 * pw, 0.0, w)
    y2 = jnp.clip(pcy + 0.5 * ph, 0.0, h)

    flat = (jax.lax.broadcasted_iota(jnp.int32, (_SUB, _LN), 0) * _LN
            + jax.lax.broadcasted_iota(jnp.int32, (_SUB, _LN), 1))
    real = flat < _N
    keep = real & (sc > _SCORE_THRESH) & (x2 - x1 >= 1e-2) & (y2 - y1 >= 1e-2)
    pad_or_reject = jnp.where(real, -1.0, _NEG_INF)
    msc = jnp.where(keep, sc, pad_or_reject)

    offmul = jnp.maximum(h, w) + 1.0
    cls_f = (jax.lax.broadcasted_iota(jnp.int32, (_CF, _SUB, _LN), 0)
             .astype(jnp.float32) + 1.0)
    off = cls_f * offmul
    ox1 = x1 + off
    oy1 = y1 + off
    ox2 = x2 + off
    oy2 = y2 + off
    areas = (ox2 - ox1) * (oy2 - oy1)

    ms_ref[...] = msc
    ox1_ref[...] = ox1
    oy1_ref[...] = oy1
    ox2_ref[...] = ox2
    oy2_ref[...] = oy2
    ar_ref[...] = areas

    cm0 = jnp.max(jnp.max(msc, axis=2), axis=1).reshape(1, _CF)
    cm_ref[...] = cm0

    obf_ref[...] = jnp.zeros((8, 128), jnp.float32)
    olab_ref[...] = jnp.zeros((1, 128), jnp.int32)

    ci = jax.lax.broadcasted_iota(jnp.int32, (1, _CF), 1)
    li = (jax.lax.broadcasted_iota(jnp.int32, (1, _SUB, _LN), 1) * _LN
          + jax.lax.broadcasted_iota(jnp.int32, (1, _SUB, _LN), 2))
    lo = jax.lax.broadcasted_iota(jnp.int32, (1, 128), 1)
    big_i = jnp.int32(1 << 30)

    def _red3(x, op):
        return op(op(x, axis=2, keepdims=True), axis=1, keepdims=True)

    mval0 = jnp.max(cm0, axis=1, keepdims=True)
    cstar0 = jnp.min(jnp.where(cm0 == mval0, ci, big_i))

    def step(t, carry):
        cstar, mval = carry

        cm = cm_ref[...]
        cmx = jnp.where(ci == cstar, _NEG_INF, cm)
        rv = jnp.max(cmx, axis=1, keepdims=True)
        ri = jnp.min(jnp.where(cmx == rv, ci, big_i))

        mval3 = mval.reshape(1, 1, 1)
        srow = ms_ref[pl.ds(cstar, 1), :, :]
        istar = _red3(jnp.where(srow == mval3, li, big_i), jnp.min)
        onehot = li == istar

        x1r = ox1_ref[pl.ds(cstar, 1), :, :]
        y1r = oy1_ref[pl.ds(cstar, 1), :, :]
        x2r = ox2_ref[pl.ds(cstar, 1), :, :]
        y2r = oy2_ref[pl.ds(cstar, 1), :, :]
        arow = ar_ref[pl.ds(cstar, 1), :, :]

        zf = jnp.float32(0.0)
        cx1 = _red3(jnp.where(onehot, x1r, zf), jnp.sum)
        cy1 = _red3(jnp.where(onehot, y1r, zf), jnp.sum)
        cx2 = _red3(jnp.where(onehot, x2r, zf), jnp.sum)
        cy2 = _red3(jnp.where(onehot, y2r, zf), jnp.sum)
        area1 = _red3(jnp.where(onehot, arow, zf), jnp.sum)

        iw = jnp.maximum(jnp.minimum(cx2, x2r) - jnp.maximum(cx1, x1r), 0.0)
        ih = jnp.maximum(jnp.minimum(cy2, y2r) - jnp.maximum(cy1, y1r), 0.0)
        inter = iw * ih
        iou = inter / (area1 + arow - inter + 1e-9)
        suppress = (iou > _NMS_THRESH) | onehot
        newrow = jnp.where(suppress, _NEG_INF, srow)
        ms_ref[pl.ds(cstar, 1), :, :] = newrow

        nm = _red3(newrow, jnp.max).reshape(1, 1)
        cm_ref[...] = jnp.where(ci == cstar, nm, cm)

        valid = mval > 0.0
        offc = (cstar.astype(jnp.float32) + 1.0) * offmul
        oh_t = lo == t
        vals = (
            jnp.where(valid, cx1.reshape(1, 1) - offc, 0.0),
            jnp.where(valid, cy1.reshape(1, 1) - offc, 0.0),
            jnp.where(valid, cx2.reshape(1, 1) - offc, 0.0),
            jnp.where(valid, cy2.reshape(1, 1) - offc, 0.0),
            jnp.where(valid, mval, 0.0),
        )
        for r, v in enumerate(vals):
            obf_ref[r:r + 1, :] = jnp.where(oh_t, v, obf_ref[r:r + 1, :])
        lab = jnp.where(valid, cstar + 1, 0)
        olab_ref[...] = jnp.where(oh_t, lab, olab_ref[...])

        take_cur = (nm > rv).astype(jnp.int32)[0, 0]
        mval2 = jnp.where(nm > rv, nm, rv)
        cstar2 = jnp.where(take_cur == 1, cstar, ri)
        return (cstar2, mval2)

    jax.lax.fori_loop(0, _K, step, (cstar0, mval0))


def _tblk(x):
    cols = x.shape[1]
    return (jnp.pad(x, ((0, _NP - _N), (0, 0)))
            .reshape(_SUB, _LN, cols).transpose(2, 0, 1))


def kernel(class_logits, box_regression, proposals, image_shape):
    lg = _tblk(class_logits)
    br = _tblk(box_regression).reshape(_C, 4, _SUB, _LN)
    pr = _tblk(proposals)
    hw = image_shape.astype(jnp.float32).reshape(1, 2)

    obf, olab = pl.pallas_call(
        _nms_body,
        out_shape=[
            jax.ShapeDtypeStruct((8, 128), jnp.float32),
            jax.ShapeDtypeStruct((1, 128), jnp.int32),
        ],
        in_specs=[
            pl.BlockSpec(memory_space=pltpu.VMEM),
            pl.BlockSpec(memory_space=pltpu.VMEM),
            pl.BlockSpec(memory_space=pltpu.VMEM),
            pl.BlockSpec(memory_space=pltpu.SMEM),
        ],
        out_specs=[
            pl.BlockSpec(memory_space=pltpu.VMEM),
            pl.BlockSpec(memory_space=pltpu.VMEM),
        ],
        scratch_shapes=[
            pltpu.VMEM((_CF, _SUB, _LN), jnp.float32),
            pltpu.VMEM((_CF, _SUB, _LN), jnp.float32),
            pltpu.VMEM((_CF, _SUB, _LN), jnp.float32),
            pltpu.VMEM((_CF, _SUB, _LN), jnp.float32),
            pltpu.VMEM((_CF, _SUB, _LN), jnp.float32),
            pltpu.VMEM((_CF, _SUB, _LN), jnp.float32),
            pltpu.VMEM((1, _CF), jnp.float32),
        ],
        compiler_params=pltpu.CompilerParams(
            vmem_limit_bytes=100 * 1024 * 1024,
        ),
    )(lg, br, pr, hw)

    boxes = obf[:4, :_K].T
    scores = obf[4, :_K]
    labels = olab[0, :_K]
    return boxes, scores, labels


# speedup vs baseline: 1.2189x; 1.2189x over previous
import math

import jax
import jax.numpy as jnp
from jax.experimental import pallas as pl
from jax.experimental.pallas import tpu as pltpu

_N = 5000
_C = 91
_CF = _C - 1
_CC = _C * 4
_K = 100
_SUB = 8
_LN = 640
_LAST = _N - (_SUB - 1) * _LN
_CLIP = math.log(1000.0 / 16.0)
_SCORE_THRESH = 0.05
_NMS_THRESH = 0.5
_NEG_INF = float("-inf")


def _transpose_blocks(src_ref, dst_ref):
    for s in range(_SUB):
        if s < _SUB - 1:
            blk = src_ref[s * _LN:(s + 1) * _LN, :]
            dst_ref[:, s, :] = jnp.transpose(blk, (1, 0))
        else:
            blk = src_ref[s * _LN:_N, :]
            dst_ref[:, s, 0:_LAST] = jnp.transpose(blk, (1, 0))


def _nms_body(lg_ref, br_ref, pr_ref, hw_ref,
              obf_ref, olab_ref,
              lgt_ref, brt_ref, prt_ref, ms_ref, xy1_ref, xy2_ref, cm_ref):
    h = hw_ref[0, 0]
    w = hw_ref[0, 1]

    _transpose_blocks(lg_ref, lgt_ref)
    _transpose_blocks(br_ref, brt_ref)
    _transpose_blocks(pr_ref, prt_ref)

    px1 = prt_ref[0]
    py1 = prt_ref[1]
    px2 = prt_ref[2]
    py2 = prt_ref[3]
    widths = px2 - px1
    heights = py2 - py1
    ctr_x = px1 + 0.5 * widths
    ctr_y = py1 + 0.5 * heights

    l90 = lgt_ref[1:_C]
    l0 = lgt_ref[0]
    mx = jnp.maximum(jnp.max(l90, axis=0), l0)
    e90 = jnp.exp(l90 - mx)
    denom = jnp.sum(e90, axis=0) + jnp.exp(l0 - mx)
    sc = e90 / denom

    kid = jax.lax.broadcasted_iota(jnp.int32, (_CC, 1, 1), 0)
    k2 = kid & 3
    parity = kid & 1
    brv = brt_ref[...]
    scaled = brv / jnp.where(k2 < 2, 10.0, 5.0)
    clipped = jnp.where(k2 >= 2, jnp.minimum(scaled, _CLIP), scaled)
    size = jnp.where(parity == 0, widths, heights)
    ctr = jnp.where(parity == 0, ctr_x, ctr_y)
    pctr = clipped * size + ctr
    psz = jnp.exp(clipped) * size
    psz_sh = jnp.concatenate([psz[2:], psz[0:2]], axis=0)
    bound = jnp.where(parity == 0, w, h)
    ux1 = jnp.clip(pctr - 0.5 * psz_sh, 0.0, bound)
    ux2 = jnp.clip(pctr + 0.5 * psz_sh, 0.0, bound)
    offmul = jnp.maximum(h, w) + 1.0
    off = (kid >> 2).astype(jnp.float32) * offmul
    xy1_ref[...] = ux1 + off
    xy2_ref[...] = ux2 + off
    szv = ux2 - ux1

    xs = jnp.concatenate([szv[4 * j + 4:4 * j + 5] for j in range(_CF)], 0)
    ys = jnp.concatenate([szv[4 * j + 5:4 * j + 6] for j in range(_CF)], 0)

    flat = (jax.lax.broadcasted_iota(jnp.int32, (_SUB, _LN), 0) * _LN
            + jax.lax.broadcasted_iota(jnp.int32, (_SUB, _LN), 1))
    real = flat < _N
    keep = real & (sc > _SCORE_THRESH) & (xs >= 1e-2) & (ys >= 1e-2)
    pad_or_reject = jnp.where(real, -1.0, _NEG_INF)
    msc = jnp.where(keep, sc, pad_or_reject)
    ms_ref[...] = msc

    cm0 = jnp.max(jnp.max(msc, axis=2), axis=1).reshape(1, _CF)
    cm_ref[...] = cm0

    obf_ref[...] = jnp.zeros((8, 128), jnp.float32)
    olab_ref[...] = jnp.zeros((1, 128), jnp.int32)

    ci = jax.lax.broadcasted_iota(jnp.int32, (1, _CF), 1)
    li = (jax.lax.broadcasted_iota(jnp.int32, (1, _SUB, _LN), 1) * _LN
          + jax.lax.broadcasted_iota(jnp.int32, (1, _SUB, _LN), 2))
    lo = jax.lax.broadcasted_iota(jnp.int32, (1, 128), 1)
    big_i = jnp.int32(1 << 30)

    def _red3(x, op):
        return op(op(x, axis=2, keepdims=True), axis=1, keepdims=True)

    mval0 = jnp.max(cm0, axis=1, keepdims=True)
    cstar0 = jnp.min(jnp.where(cm0 == mval0, ci, big_i))

    def step(t, carry):
        cstar, mval = carry

        cm = cm_ref[...]
        cmx = jnp.where(ci == cstar, _NEG_INF, cm)
        rv = jnp.max(cmx, axis=1, keepdims=True)
        ri = jnp.min(jnp.where(cmx == rv, ci, big_i))

        mval3 = mval.reshape(1, 1, 1)
        srow = ms_ref[pl.ds(cstar, 1), :, :]
        istar = _red3(jnp.where(srow == mval3, li, big_i), jnp.min)
        onehot = li == istar

        b = cstar * 4 + 4
        x1r = xy1_ref[pl.ds(b, 1), :, :]
        y1r = xy1_ref[pl.ds(b + 1, 1), :, :]
        x2r = xy2_ref[pl.ds(b, 1), :, :]
        y2r = xy2_ref[pl.ds(b + 1, 1), :, :]
        arow = (x2r - x1r) * (y2r - y1r)

        zf = jnp.float32(0.0)
        cx1 = _red3(jnp.where(onehot, x1r, zf), jnp.sum)
        cy1 = _red3(jnp.where(onehot, y1r, zf), jnp.sum)
        cx2 = _red3(jnp.where(onehot, x2r, zf), jnp.sum)
        cy2 = _red3(jnp.where(onehot, y2r, zf), jnp.sum)
        area1 = (cx2 - cx1) * (cy2 - cy1)

        iw = jnp.maximum(jnp.minimum(cx2, x2r) - jnp.maximum(cx1, x1r), 0.0)
        ih = jnp.maximum(jnp.minimum(cy2, y2r) - jnp.maximum(cy1, y1r), 0.0)
        inter = iw * ih
        iou = inter / (area1 + arow - inter + 1e-9)
        suppress = (iou > _NMS_THRESH) | onehot
        newrow = jnp.where(suppress, _NEG_INF, srow)
        ms_ref[pl.ds(cstar, 1), :, :] = newrow

        nm = _red3(newrow, jnp.max).reshape(1, 1)
        cm_ref[...] = jnp.where(ci == cstar, nm, cm)

        valid = mval > 0.0
        offc = (cstar.astype(jnp.float32) + 1.0) * offmul
        oh_t = lo == t
        vals = (
            jnp.where(valid, cx1.reshape(1, 1) - offc, 0.0),
            jnp.where(valid, cy1.reshape(1, 1) - offc, 0.0),
            jnp.where(valid, cx2.reshape(1, 1) - offc, 0.0),
            jnp.where(valid, cy2.reshape(1, 1) - offc, 0.0),
            jnp.where(valid, mval, 0.0),
        )
        for r, v in enumerate(vals):
            obf_ref[r:r + 1, :] = jnp.where(oh_t, v, obf_ref[r:r + 1, :])
        lab = jnp.where(valid, cstar + 1, 0)
        olab_ref[...] = jnp.where(oh_t, lab, olab_ref[...])

        take_cur = (nm > rv).astype(jnp.int32)[0, 0]
        mval2 = jnp.where(nm > rv, nm, rv)
        cstar2 = jnp.where(take_cur == 1, cstar, ri)
        return (cstar2, mval2)

    jax.lax.fori_loop(0, _K, step, (cstar0, mval0))


def kernel(class_logits, box_regression, proposals, image_shape):
    hw = image_shape.astype(jnp.float32).reshape(1, 2)

    obf, olab = pl.pallas_call(
        _nms_body,
        out_shape=[
            jax.ShapeDtypeStruct((8, 128), jnp.float32),
            jax.ShapeDtypeStruct((1, 128), jnp.int32),
        ],
        in_specs=[
            pl.BlockSpec(memory_space=pltpu.VMEM),
            pl.BlockSpec(memory_space=pltpu.VMEM),
            pl.BlockSpec(memory_space=pltpu.VMEM),
            pl.BlockSpec(memory_space=pltpu.SMEM),
        ],
        out_specs=[
            pl.BlockSpec(memory_space=pltpu.VMEM),
            pl.BlockSpec(memory_space=pltpu.VMEM),
        ],
        scratch_shapes=[
            pltpu.VMEM((_C, _SUB, _LN), jnp.float32),
            pltpu.VMEM((_CC, _SUB, _LN), jnp.float32),
            pltpu.VMEM((4, _SUB, _LN), jnp.float32),
            pltpu.VMEM((_CF, _SUB, _LN), jnp.float32),
            pltpu.VMEM((_CC, _SUB, _LN), jnp.float32),
            pltpu.VMEM((_CC, _SUB, _LN), jnp.float32),
            pltpu.VMEM((1, _CF), jnp.float32),
        ],
        compiler_params=pltpu.CompilerParams(
            vmem_limit_bytes=128 * 1024 * 1024,
        ),
    )(class_logits, box_regression, proposals, hw)

    boxes = obf[:4, :_K].T
    scores = obf[4, :_K]
    labels = olab[0, :_K]
    return boxes, scores, labels


# speedup vs baseline: 1.2198x; 1.0007x over previous
import math

import jax
import jax.numpy as jnp
from jax.experimental import pallas as pl
from jax.experimental.pallas import tpu as pltpu

_N = 5000
_C = 91
_CF = _C - 1
_CC = _C * 4
_K = 100
_SUB = 8
_LN = 640
_LAST = _N - (_SUB - 1) * _LN
_CLIP = math.log(1000.0 / 16.0)
_SCORE_THRESH = 0.05
_NMS_THRESH = 0.5
_NEG_INF = float("-inf")


def _transpose_blocks(src_ref, dst_ref):
    for s in range(_SUB):
        if s < _SUB - 1:
            blk = src_ref[s * _LN:(s + 1) * _LN, :]
            dst_ref[:, s, :] = jnp.transpose(blk, (1, 0))
        else:
            blk = src_ref[s * _LN:_N, :]
            dst_ref[:, s, 0:_LAST] = jnp.transpose(blk, (1, 0))


def _nms_body(lg_ref, br_ref, pr_ref, hw_ref,
              obf_ref, olab_ref,
              lgt_ref, brt_ref, prt_ref, ms_ref, xy1_ref, xy2_ref, cm_ref):
    h = hw_ref[0, 0]
    w = hw_ref[0, 1]

    _transpose_blocks(lg_ref, lgt_ref)
    _transpose_blocks(br_ref, brt_ref)
    _transpose_blocks(pr_ref, prt_ref)

    px1 = prt_ref[0]
    py1 = prt_ref[1]
    px2 = prt_ref[2]
    py2 = prt_ref[3]
    widths = px2 - px1
    heights = py2 - py1
    ctr_x = px1 + 0.5 * widths
    ctr_y = py1 + 0.5 * heights

    l90 = lgt_ref[1:_C]
    l0 = lgt_ref[0]
    mx = jnp.maximum(jnp.max(l90, axis=0), l0)
    e90 = jnp.exp(l90 - mx)
    denom = jnp.sum(e90, axis=0) + jnp.exp(l0 - mx)
    sc = e90 / denom

    kid = jax.lax.broadcasted_iota(jnp.int32, (_CC, 1, 1), 0)
    k2 = kid & 3
    parity = kid & 1
    brv = brt_ref[...]
    scaled = brv / jnp.where(k2 < 2, 10.0, 5.0)
    clipped = jnp.where(k2 >= 2, jnp.minimum(scaled, _CLIP), scaled)
    size = jnp.where(parity == 0, widths, heights)
    ctr = jnp.where(parity == 0, ctr_x, ctr_y)
    pctr = clipped * size + ctr
    psz = jnp.exp(clipped) * size
    psz_sh = jnp.concatenate([psz[2:], psz[0:2]], axis=0)
    bound = jnp.where(parity == 0, w, h)
    ux1 = jnp.clip(pctr - 0.5 * psz_sh, 0.0, bound)
    ux2 = jnp.clip(pctr + 0.5 * psz_sh, 0.0, bound)
    offmul = jnp.maximum(h, w) + 1.0
    off = (kid >> 2).astype(jnp.float32) * offmul
    xy1_ref[...] = ux1 + off
    xy2_ref[...] = ux2 + off
    szv = ux2 - ux1

    xs = jnp.concatenate([szv[4 * j + 4:4 * j + 5] for j in range(_CF)], 0)
    ys = jnp.concatenate([szv[4 * j + 5:4 * j + 6] for j in range(_CF)], 0)

    flat = (jax.lax.broadcasted_iota(jnp.int32, (_SUB, _LN), 0) * _LN
            + jax.lax.broadcasted_iota(jnp.int32, (_SUB, _LN), 1))
    real = flat < _N
    keep = real & (sc > _SCORE_THRESH) & (xs >= 1e-2) & (ys >= 1e-2)
    pad_or_reject = jnp.where(real, -1.0, _NEG_INF)
    msc = jnp.where(keep, sc, pad_or_reject)
    ms_ref[...] = msc

    cm0 = jnp.max(jnp.max(msc, axis=2), axis=1).reshape(1, _CF)
    cm_ref[...] = cm0

    obf_ref[...] = jnp.zeros((8, 128), jnp.float32)
    olab_ref[...] = jnp.zeros((1, 128), jnp.int32)

    ci = jax.lax.broadcasted_iota(jnp.int32, (1, _CF), 1)
    li = (jax.lax.broadcasted_iota(jnp.int32, (1, _SUB, _LN), 1) * _LN
          + jax.lax.broadcasted_iota(jnp.int32, (1, _SUB, _LN), 2))
    lo = jax.lax.broadcasted_iota(jnp.int32, (1, 128), 1)
    big_i = jnp.int32(1 << 30)

    def _red3(x, op):
        return op(op(x, axis=2, keepdims=True), axis=1, keepdims=True)

    mval0 = jnp.max(cm0, axis=1, keepdims=True)
    cstar0 = jnp.min(jnp.where(cm0 == mval0, ci, big_i))

    def step(t, carry):
        cstar, mval = carry

        cm = cm_ref[...]
        cmx = jnp.where(ci == cstar, _NEG_INF, cm)
        rv = jnp.max(cmx, axis=1, keepdims=True)
        ri = jnp.min(jnp.where(cmx == rv, ci, big_i))

        mval3 = mval.reshape(1, 1, 1)
        srow = ms_ref[pl.ds(cstar, 1), :, :]
        istar = _red3(jnp.where(srow == mval3, li, big_i), jnp.min)
        onehot = li == istar

        b = cstar * 4 + 4
        x1r = xy1_ref[pl.ds(b, 1), :, :]
        y1r = xy1_ref[pl.ds(b + 1, 1), :, :]
        x2r = xy2_ref[pl.ds(b, 1), :, :]
        y2r = xy2_ref[pl.ds(b + 1, 1), :, :]
        arow = (x2r - x1r) * (y2r - y1r)

        zf = jnp.float32(0.0)
        cx1 = _red3(jnp.where(onehot, x1r, zf), jnp.sum)
        cy1 = _red3(jnp.where(onehot, y1r, zf), jnp.sum)
        cx2 = _red3(jnp.where(onehot, x2r, zf), jnp.sum)
        cy2 = _red3(jnp.where(onehot, y2r, zf), jnp.sum)
        area1 = (cx2 - cx1) * (cy2 - cy1)

        iw = jnp.maximum(jnp.minimum(cx2, x2r) - jnp.maximum(cx1, x1r), 0.0)
        ih = jnp.maximum(jnp.minimum(cy2, y2r) - jnp.maximum(cy1, y1r), 0.0)
        inter = iw * ih
        iou = inter / (area1 + arow - inter + 1e-9)
        suppress = (iou > _NMS_THRESH) | onehot
        newrow = jnp.where(suppress, _NEG_INF, srow)
        ms_ref[pl.ds(cstar, 1), :, :] = newrow

        nm = _red3(newrow, jnp.max).reshape(1, 1)
        cm_ref[...] = jnp.where(ci == cstar, nm, cm)

        valid = mval > 0.0
        offc = (cstar.astype(jnp.float32) + 1.0) * offmul
        oh_t = lo == t
        vals = (
            jnp.where(valid, cx1.reshape(1, 1) - offc, 0.0),
            jnp.where(valid, cy1.reshape(1, 1) - offc, 0.0),
            jnp.where(valid, cx2.reshape(1, 1) - offc, 0.0),
            jnp.where(valid, cy2.reshape(1, 1) - offc, 0.0),
            jnp.where(valid, mval, 0.0),
        )
        for r, v in enumerate(vals):
            obf_ref[r:r + 1, :] = jnp.where(oh_t, v, obf_ref[r:r + 1, :])
        lab = jnp.where(valid, cstar + 1, 0)
        olab_ref[...] = jnp.where(oh_t, lab, olab_ref[...])

        take_cur = (nm > rv).astype(jnp.int32)[0, 0]
        mval2 = jnp.where(nm > rv, nm, rv)
        cstar2 = jnp.where(take_cur == 1, cstar, ri)
        return (cstar2, mval2)

    def step2(i, carry):
        return step(2 * i + 1, step(2 * i, carry))

    jax.lax.fori_loop(0, _K // 2, step2, (cstar0, mval0))


def kernel(class_logits, box_regression, proposals, image_shape):
    hw = image_shape.astype(jnp.float32).reshape(1, 2)

    obf, olab = pl.pallas_call(
        _nms_body,
        out_shape=[
            jax.ShapeDtypeStruct((8, 128), jnp.float32),
            jax.ShapeDtypeStruct((1, 128), jnp.int32),
        ],
        in_specs=[
            pl.BlockSpec(memory_space=pltpu.VMEM),
            pl.BlockSpec(memory_space=pltpu.VMEM),
            pl.BlockSpec(memory_space=pltpu.VMEM),
            pl.BlockSpec(memory_space=pltpu.SMEM),
        ],
        out_specs=[
            pl.BlockSpec(memory_space=pltpu.VMEM),
            pl.BlockSpec(memory_space=pltpu.VMEM),
        ],
        scratch_shapes=[
            pltpu.VMEM((_C, _SUB, _LN), jnp.float32),
            pltpu.VMEM((_CC, _SUB, _LN), jnp.float32),
            pltpu.VMEM((4, _SUB, _LN), jnp.float32),
            pltpu.VMEM((_CF, _SUB, _LN), jnp.float32),
            pltpu.VMEM((_CC, _SUB, _LN), jnp.float32),
            pltpu.VMEM((_CC, _SUB, _LN), jnp.float32),
            pltpu.VMEM((1, _CF), jnp.float32),
        ],
        compiler_params=pltpu.CompilerParams(
            vmem_limit_bytes=128 * 1024 * 1024,
        ),
    )(class_logits, box_regression, proposals, hw)

    boxes = obf[:4, :_K].T
    scores = obf[4, :_K]
    labels = olab[0, :_K]
    return boxes, scores, labels
